# Initial kernel scaffold; baseline (speedup 1.0000x reference)
import jax
import jax.numpy as jnp
from jax.experimental import pallas as pl


def kernel(inputs, token_table, pos_table):
    raise NotImplementedError("write your pallas kernel here")


# speedup vs baseline: 3.2905x; 3.2905x over previous
import jax
import jax.numpy as jnp
from jax import lax
from jax.experimental import pallas as pl
from jax.experimental.pallas import tpu as pltpu
from jax.experimental.pallas import tpu_sc as plsc

BATCH = 4096
SEQ = 200
EMBED = 64
NC, NS, LANES = 2, 16, 16
NW = NC * NS
ROWS_PER_W = BATCH // NW
IDX_PER_W = ROWS_PER_W * SEQ


def _body(idx_hbm, tok_hbm, pos_hbm, out_hbm, idx_v, pos_v, buf_v, gsem):
    wid = lax.axis_index("s") * NC + lax.axis_index("c")
    base = wid * IDX_PER_W
    pltpu.sync_copy(idx_hbm.at[pl.ds(base, IDX_PER_W)], idx_v)
    pltpu.sync_copy(pos_hbm, pos_v)

    @pl.loop(0, ROWS_PER_W)
    def _chunk(k):
        off = k * SEQ
        pltpu.async_copy(
            tok_hbm.at[idx_v.at[pl.ds(off, SEQ)]], buf_v, gsem
        ).wait()

        @pl.loop(0, SEQ)
        def _addrow(r):
            for c in range(EMBED // LANES):
                s = pl.ds(c * LANES, LANES)
                buf_v[r, s] = buf_v[r, s] + pos_v[r, s]

        pltpu.sync_copy(buf_v, out_hbm.at[pl.ds(base + off, SEQ)])


def kernel(inputs, token_table, pos_table):
    flat_idx = inputs.reshape(-1).astype(jnp.int32)
    mesh = plsc.VectorSubcoreMesh(core_axis_name="c", subcore_axis_name="s")
    out = pl.kernel(
        _body,
        out_type=jax.ShapeDtypeStruct((BATCH * SEQ, EMBED), jnp.float32),
        mesh=mesh,
        scratch_types=[
            pltpu.VMEM((IDX_PER_W,), jnp.int32),
            pltpu.VMEM((SEQ, EMBED), jnp.float32),
            pltpu.VMEM((SEQ, EMBED), jnp.float32),
            pltpu.SemaphoreType.DMA,
        ],
        compiler_params=pltpu.CompilerParams(use_tc_tiling_on_sc=False),
    )(flat_idx, token_table, pos_table)
    return out.reshape(BATCH, SEQ, EMBED)


# speedup vs baseline: 3.4850x; 1.0591x over previous
import jax
import jax.numpy as jnp
from jax import lax
from jax.experimental import pallas as pl
from jax.experimental.pallas import tpu as pltpu
from jax.experimental.pallas import tpu_sc as plsc

BATCH = 4096
SEQ = 200
EMBED = 64
NC, NS, LANES = 2, 16, 16
NW = NC * NS
ROWS_PER_W = BATCH // NW
IDX_PER_W = ROWS_PER_W * SEQ
NBUF = 4


def _body(idx_hbm, tok_hbm, pos_hbm, out_hbm, idx_v, pos_v, bufs, gsems, osems):
    wid = lax.axis_index("s") * NC + lax.axis_index("c")
    base = wid * IDX_PER_W
    pltpu.sync_copy(idx_hbm.at[pl.ds(base, IDX_PER_W)], idx_v)
    pltpu.sync_copy(pos_hbm, pos_v)

    def gather(j, b):
        pltpu.async_copy(
            tok_hbm.at[idx_v.at[pl.ds(j * SEQ, SEQ)]], bufs.at[b], gsems[b]
        )

    def gather_wait(j, b):
        pltpu.make_async_copy(
            tok_hbm.at[idx_v.at[pl.ds(j * SEQ, SEQ)]], bufs.at[b], gsems[b]
        ).wait()

    def put(j, b):
        pltpu.async_copy(bufs.at[b], out_hbm.at[pl.ds(base + j * SEQ, SEQ)], osems[b])

    def put_wait(j, b):
        pltpu.make_async_copy(
            bufs.at[b], out_hbm.at[pl.ds(base + j * SEQ, SEQ)], osems[b]
        ).wait()

    for b in range(NBUF - 1):
        gather(b, b)

    @pl.loop(0, ROWS_PER_W, step=NBUF)
    def _outer(k0):
        for b in range(NBUF):
            j = k0 + b
            gather_wait(j, b)

            @pl.loop(0, SEQ, unroll=4)
            def _addrow(r):
                for c in range(EMBED // LANES):
                    s = pl.ds(c * LANES, LANES)
                    bufs[b, r, s] = bufs[b, r, s] + pos_v[r, s]

            put(j, b)
            jn = j + NBUF - 1
            bn = (b + NBUF - 1) % NBUF

            @pl.when(jn < ROWS_PER_W)
            def _prefetch():
                @pl.when(j >= 1)
                def _drain_prev():
                    put_wait(j - 1, bn)

                gather(jn, bn)

    for b in range(NBUF):
        put_wait(ROWS_PER_W - NBUF + b, (ROWS_PER_W - NBUF + b) % NBUF)


def kernel(inputs, token_table, pos_table):
    flat_idx = inputs.reshape(-1).astype(jnp.int32)
    mesh = plsc.VectorSubcoreMesh(core_axis_name="c", subcore_axis_name="s")
    out = pl.kernel(
        _body,
        out_type=jax.ShapeDtypeStruct((BATCH * SEQ, EMBED), jnp.float32),
        mesh=mesh,
        scratch_types=[
            pltpu.VMEM((IDX_PER_W,), jnp.int32),
            pltpu.VMEM((SEQ, EMBED), jnp.float32),
            pltpu.VMEM((NBUF, SEQ, EMBED), jnp.float32),
            [pltpu.SemaphoreType.DMA] * NBUF,
            [pltpu.SemaphoreType.DMA] * NBUF,
        ],
        compiler_params=pltpu.CompilerParams(use_tc_tiling_on_sc=False),
    )(flat_idx, token_table, pos_table)
    return out.reshape(BATCH, SEQ, EMBED)
